# baseline (speedup 1.0000x reference)
import jax
import jax.numpy as jnp
from jax import lax
from jax.experimental import pallas as pl
from jax.experimental.pallas import tpu as pltpu
from jax.experimental.pallas import tpu_sc as plsc

DIM = 33
TSIZE = DIM * DIM * DIM
CPAD = 35944
NC, NS, L = 2, 16, 16
NW = NC * NS
B, C, H, W = 8, 3, 512, 512
PLANE = H * W
PIX = B * PLANE
PER_TILE = PIX // NW
TILES_PER_IMG = PLANE // PER_TILE
CHUNK = 1024
NCHUNK = PER_TILE // CHUNK
NGRP = CHUNK // L
FMAX = float(DIM - 1)


def _body(lut_hbm, img_hbm, out_hbm,
          l0, l1, l2,
          ir0, ig0, ib0, ir1, ig1, ib1,
          or0, og0, ob0, or1, og1, ob1,
          si0, si1, so0, so1):
    wid = lax.axis_index("s") * NC + lax.axis_index("c")
    b3 = (wid // TILES_PER_IMG) * 3
    hw0 = (wid % TILES_PER_IMG) * PER_TILE

    pltpu.sync_copy(lut_hbm.at[pl.ds(0, CPAD)], l0)
    pltpu.sync_copy(lut_hbm.at[pl.ds(CPAD, CPAD)], l1)
    pltpu.sync_copy(lut_hbm.at[pl.ds(2 * CPAD, CPAD)], l2)

    in_slots = ((ir0, ig0, ib0, si0), (ir1, ig1, ib1, si1))
    out_slots = ((or0, og0, ob0, so0), (or1, og1, ob1, so1))

    def chunk_base(i):
        return b3 * PLANE + hw0 + i * CHUNK

    def start_in(i, s):
        base = chunk_base(i)
        r, g, b_, sem = in_slots[s]
        pltpu.async_copy(img_hbm.at[pl.ds(base, CHUNK)], r, sem)
        pltpu.async_copy(img_hbm.at[pl.ds(base + PLANE, CHUNK)], g, sem)
        pltpu.async_copy(img_hbm.at[pl.ds(base + 2 * PLANE, CHUNK)], b_, sem)

    def wait_in(s):
        r, g, b_, sem = in_slots[s]
        for dst in (r, g, b_):
            pltpu.make_async_copy(img_hbm.at[pl.ds(0, CHUNK)], dst, sem).wait()

    def start_out(i, s):
        base = chunk_base(i)
        o0, o1, o2, sem = out_slots[s]
        pltpu.async_copy(o0, out_hbm.at[pl.ds(base, CHUNK)], sem)
        pltpu.async_copy(o1, out_hbm.at[pl.ds(base + PLANE, CHUNK)], sem)
        pltpu.async_copy(o2, out_hbm.at[pl.ds(base + 2 * PLANE, CHUNK)], sem)

    def wait_out(s):
        o0, o1, o2, sem = out_slots[s]
        for src in (o0, o1, o2):
            pltpu.make_async_copy(src, out_hbm.at[pl.ds(0, CHUNK)], sem).wait()

    def compute_chunk(s):
        inr, ing, inb, _ = in_slots[s]
        outs = out_slots[s][:3]

        @plsc.parallel_loop(0, NGRP, unroll=2)
        def _grp(j):
            sl = pl.ds(j * L, L)

            def axis(ref):
                cf = jnp.minimum(jnp.maximum(ref[sl] * FMAX, 0.0), FMAX)
                i0 = cf.astype(jnp.int32)
                w = cf - i0.astype(jnp.float32)
                i1 = jnp.minimum(i0 + 1, DIM - 1)
                return i0, i1, w

            x0, x1, wx = axis(inr)
            y0, y1, wy = axis(ing)
            z0, z1, wz = axis(inb)
            tz0 = z0 * (DIM * DIM)
            tz1 = z1 * (DIM * DIM)
            uy0 = y0 * DIM
            uy1 = y1 * DIM
            zy00 = tz0 + uy0
            zy01 = tz0 + uy1
            zy10 = tz1 + uy0
            zy11 = tz1 + uy1
            ii = (zy00 + x0, zy00 + x1, zy01 + x0, zy01 + x1,
                  zy10 + x0, zy10 + x1, zy11 + x0, zy11 + x1)
            cx = 1.0 - wx
            cy = 1.0 - wy
            cz = 1.0 - wz
            q00 = cz * cy
            q01 = cz * wy
            q10 = wz * cy
            q11 = wz * wy
            wts = (q00 * cx, q00 * wx, q01 * cx, q01 * wx,
                   q10 * cx, q10 * wx, q11 * cx, q11 * wx)
            for lc, o in ((l0, outs[0]), (l1, outs[1]), (l2, outs[2])):
                acc_a = plsc.load_gather(lc, [ii[0]]) * wts[0]
                acc_b = plsc.load_gather(lc, [ii[1]]) * wts[1]
                for k in range(2, 8, 2):
                    acc_a = acc_a + plsc.load_gather(lc, [ii[k]]) * wts[k]
                    acc_b = acc_b + plsc.load_gather(lc, [ii[k + 1]]) * wts[k + 1]
                o[sl] = acc_a + acc_b

    start_in(0, 0)
    start_in(1, 1)

    @pl.loop(0, NCHUNK, step=2)
    def _chunk(i):
        for s in range(2):
            ci = i + s
            wait_in(s)

            @pl.when(ci >= 2)
            def _():
                wait_out(s)

            compute_chunk(s)
            start_out(ci, s)

            @pl.when(ci + 2 < NCHUNK)
            def _():
                start_in(ci + 2, s)

    wait_out(0)
    wait_out(1)


_tri = pl.kernel(
    _body,
    out_type=jax.ShapeDtypeStruct((B * C * PLANE,), jnp.float32),
    mesh=plsc.VectorSubcoreMesh(
        core_axis_name="c", subcore_axis_name="s",
        num_cores=NC, num_subcores=NS),
    compiler_params=pltpu.CompilerParams(needs_layout_passes=False),
    scratch_types=(
        [pltpu.VMEM((CPAD,), jnp.float32)] * 3
        + [pltpu.VMEM((CHUNK,), jnp.float32)] * 12
        + [pltpu.SemaphoreType.DMA] * 4
    ),
)


def kernel(lut, img):
    lut_pad = jnp.pad(lut.reshape(3, TSIZE), ((0, 0), (0, CPAD - TSIZE))).reshape(-1)
    img_flat = img.reshape(-1)
    out = _tri(lut_pad, img_flat)
    return out.reshape(B, C, H, W)


# speedup vs baseline: 1.4205x; 1.4205x over previous
import jax
import jax.numpy as jnp
from jax import lax
from jax.experimental import pallas as pl
from jax.experimental.pallas import tpu as pltpu
from jax.experimental.pallas import tpu_sc as plsc

DIM = 33
TSIZE = DIM * DIM * DIM
CPAD = 35944
NC, NS, L = 2, 16, 16
NW = NC * NS
B, C, H, W = 8, 3, 512, 512
PLANE = H * W
PIX = B * PLANE
PER_TILE = PIX // NW
TILES_PER_IMG = PLANE // PER_TILE
CHUNK = 1024
NCHUNK = PER_TILE // CHUNK
NGRP = CHUNK // L
FMAX = float(DIM - 1)


def _body(lut_hbm, img_hbm, out_hbm,
          l0, l1, l2,
          ir0, ig0, ib0, ir1, ig1, ib1,
          or0, og0, ob0, or1, og1, ob1,
          si0, si1, so0, so1):
    wid = lax.axis_index("s") * NC + lax.axis_index("c")
    b3 = (wid // TILES_PER_IMG) * 3
    hw0 = (wid % TILES_PER_IMG) * PER_TILE

    pltpu.sync_copy(lut_hbm.at[pl.ds(0, CPAD)], l0)
    pltpu.sync_copy(lut_hbm.at[pl.ds(CPAD, CPAD)], l1)
    pltpu.sync_copy(lut_hbm.at[pl.ds(2 * CPAD, CPAD)], l2)

    in_slots = ((ir0, ig0, ib0, si0), (ir1, ig1, ib1, si1))
    out_slots = ((or0, og0, ob0, so0), (or1, og1, ob1, so1))

    def chunk_base(i):
        return b3 * PLANE + hw0 + i * CHUNK

    def start_in(i, s):
        base = chunk_base(i)
        r, g, b_, sem = in_slots[s]
        pltpu.async_copy(img_hbm.at[pl.ds(base, CHUNK)], r, sem)
        pltpu.async_copy(img_hbm.at[pl.ds(base + PLANE, CHUNK)], g, sem)
        pltpu.async_copy(img_hbm.at[pl.ds(base + 2 * PLANE, CHUNK)], b_, sem)

    def wait_in(s):
        r, g, b_, sem = in_slots[s]
        for dst in (r, g, b_):
            pltpu.make_async_copy(img_hbm.at[pl.ds(0, CHUNK)], dst, sem).wait()

    def start_out(i, s):
        base = chunk_base(i)
        o0, o1, o2, sem = out_slots[s]
        pltpu.async_copy(o0, out_hbm.at[pl.ds(base, CHUNK)], sem)
        pltpu.async_copy(o1, out_hbm.at[pl.ds(base + PLANE, CHUNK)], sem)
        pltpu.async_copy(o2, out_hbm.at[pl.ds(base + 2 * PLANE, CHUNK)], sem)

    def wait_out(s):
        o0, o1, o2, sem = out_slots[s]
        for src in (o0, o1, o2):
            pltpu.make_async_copy(src, out_hbm.at[pl.ds(0, CHUNK)], sem).wait()

    def compute_chunk(s):
        inr, ing, inb, _ = in_slots[s]
        outs = out_slots[s][:3]

        @plsc.parallel_loop(0, NGRP, unroll=2)
        def _grp(j):
            sl = pl.ds(j * L, L)

            def axis(ref):
                cf = jnp.minimum(jnp.maximum(ref[sl] * FMAX, 0.0), FMAX)
                i0 = cf.astype(jnp.int32)
                w = cf - i0.astype(jnp.float32)
                i1 = jnp.minimum(i0 + 1, DIM - 1)
                return i0, i1, w

            x0, x1, wx = axis(inr)
            y0, y1, wy = axis(ing)
            z0, z1, wz = axis(inb)
            tz0 = z0 * (DIM * DIM)
            tz1 = z1 * (DIM * DIM)
            uy0 = y0 * DIM
            uy1 = y1 * DIM
            zy00 = tz0 + uy0
            zy01 = tz0 + uy1
            zy10 = tz1 + uy0
            zy11 = tz1 + uy1
            cx = 1.0 - wx
            cy = 1.0 - wy
            cz = 1.0 - wz
            q00 = cz * cy
            q01 = cz * wy
            q10 = wz * cy
            q11 = wz * wy
            accs = [None, None, None]
            for zy, q in ((zy00, q00), (zy01, q01), (zy10, q10), (zy11, q11)):
                for xx, wxp in ((x0, cx), (x1, wx)):
                    idx = zy + xx
                    wk = q * wxp
                    for c, lc in enumerate((l0, l1, l2)):
                        t = plsc.load_gather(lc, [idx]) * wk
                        accs[c] = t if accs[c] is None else accs[c] + t
            for c in range(3):
                outs[c][sl] = accs[c]

    start_in(0, 0)
    start_in(1, 1)

    @pl.loop(0, NCHUNK, step=2)
    def _chunk(i):
        for s in range(2):
            ci = i + s
            wait_in(s)

            @pl.when(ci >= 2)
            def _():
                wait_out(s)

            compute_chunk(s)
            start_out(ci, s)

            @pl.when(ci + 2 < NCHUNK)
            def _():
                start_in(ci + 2, s)

    wait_out(0)
    wait_out(1)


_tri = pl.kernel(
    _body,
    out_type=jax.ShapeDtypeStruct((B * C * PLANE,), jnp.float32),
    mesh=plsc.VectorSubcoreMesh(
        core_axis_name="c", subcore_axis_name="s",
        num_cores=NC, num_subcores=NS),
    compiler_params=pltpu.CompilerParams(needs_layout_passes=False),
    scratch_types=(
        [pltpu.VMEM((CPAD,), jnp.float32)] * 3
        + [pltpu.VMEM((CHUNK,), jnp.float32)] * 12
        + [pltpu.SemaphoreType.DMA] * 4
    ),
)


def kernel(lut, img):
    lut_pad = jnp.pad(lut.reshape(3, TSIZE), ((0, 0), (0, CPAD - TSIZE))).reshape(-1)
    img_flat = img.reshape(-1)
    out = _tri(lut_pad, img_flat)
    return out.reshape(B, C, H, W)
